# baseline (speedup 1.0000x reference)
import jax
import jax.numpy as jnp
from jax.experimental import pallas as pl
from jax.experimental.pallas import tpu as pltpu

B = 4096
IN_DIMS = 1024
HID = 1024
OUT = 1024
BLK = 512
NB = B // BLK


def _body(x_ref, ws_ref, bs_ref, g0_ref, b0_ref, we_ref, be_ref, out_ref,
          h_ref, acc_ref, s_ref, t_ref):
    p = pl.program_id(0)
    i = pl.program_id(1)

    @pl.when(p == 0)
    def _phase0():
        h = jnp.dot(x_ref[...], ws_ref[...], preferred_element_type=jnp.float32)
        h = jnp.maximum(h + bs_ref[...], 0.0)
        h_ref[pl.ds(i * BLK, BLK), :] = h
        psum = jnp.sum(h, axis=0, keepdims=True)
        psq = jnp.sum(h * h, axis=0, keepdims=True)
        blk_acc = jnp.concatenate([psum, psq], axis=0)

        @pl.when(i == 0)
        def _():
            acc_ref[...] = blk_acc

        @pl.when(i > 0)
        def _():
            acc_ref[...] += blk_acc

    @pl.when(p == 1)
    def _phase1():
        @pl.when(i == 0)
        def _():
            m = acc_ref[0:1, :] / B
            v = acc_ref[1:2, :] / B - m * m
            s = g0_ref[...] * jax.lax.rsqrt(v + 1e-5)
            s_ref[...] = s
            t_ref[...] = b0_ref[...] - m * s

        h = h_ref[pl.ds(i * BLK, BLK), :]
        hn = (h * s_ref[...] + t_ref[...]).astype(jnp.bfloat16)
        o = jnp.dot(hn, we_ref[...], preferred_element_type=jnp.float32)
        out_ref[...] = jnp.maximum(o + be_ref[...], 0.0)


def kernel(x, W_start, b_start, bn0_g, bn0_b, W_exp, b_exp, bn_g, bn_b,
           W_end, b_end, W_dqn, b_dqn):
    del W_exp, b_exp, bn_g, bn_b, W_dqn, b_dqn
    bs = b_start.reshape(1, HID)
    g0 = bn0_g.reshape(1, HID)
    b0 = bn0_b.reshape(1, HID)
    be = b_end.reshape(1, OUT)
    xb = x.astype(jnp.bfloat16)
    wsb = W_start.astype(jnp.bfloat16)
    web = W_end.astype(jnp.bfloat16)
    return pl.pallas_call(
        _body,
        grid=(2, NB),
        in_specs=[
            pl.BlockSpec((BLK, IN_DIMS), lambda p, i: (i * (1 - p), 0)),
            pl.BlockSpec((IN_DIMS, HID), lambda p, i: (0, 0)),
            pl.BlockSpec((1, HID), lambda p, i: (0, 0)),
            pl.BlockSpec((1, HID), lambda p, i: (0, 0)),
            pl.BlockSpec((1, HID), lambda p, i: (0, 0)),
            pl.BlockSpec((HID, OUT), lambda p, i: (0, 0)),
            pl.BlockSpec((1, OUT), lambda p, i: (0, 0)),
        ],
        out_specs=pl.BlockSpec((BLK, OUT), lambda p, i: (i * p, 0)),
        out_shape=jax.ShapeDtypeStruct((B, OUT), jnp.float32),
        scratch_shapes=[
            pltpu.VMEM((B, HID), jnp.float32),
            pltpu.VMEM((2, HID), jnp.float32),
            pltpu.VMEM((1, HID), jnp.float32),
            pltpu.VMEM((1, HID), jnp.float32),
        ],
        compiler_params=pltpu.CompilerParams(
            dimension_semantics=("arbitrary", "arbitrary")),
    )(xb, wsb, bs, g0, b0, web, be)


# speedup vs baseline: 1.4571x; 1.4571x over previous
import jax
import jax.numpy as jnp
from jax.experimental import pallas as pl
from jax.experimental.pallas import tpu as pltpu

B = 4096
IN_DIMS = 1024
HID = 1024
OUT = 1024
BLK = 512
NB = B // BLK


def _body(x_ref, ws_ref, bs_ref, g0_ref, b0_ref, we_ref, be_ref, out_ref,
          h_ref, acc_ref, s_ref, t_ref):
    p = pl.program_id(0)
    i = pl.program_id(1)

    @pl.when(p == 0)
    def _phase0():
        h = jnp.dot(x_ref[...].astype(jnp.bfloat16),
                    ws_ref[...].astype(jnp.bfloat16),
                    preferred_element_type=jnp.float32)
        h = jnp.maximum(h + bs_ref[...], 0.0)
        h_ref[pl.ds(i * BLK, BLK), :] = h
        psum = jnp.sum(h, axis=0, keepdims=True)
        psq = jnp.sum(h * h, axis=0, keepdims=True)
        blk_acc = jnp.concatenate([psum, psq], axis=0)

        @pl.when(i == 0)
        def _():
            acc_ref[...] = blk_acc

        @pl.when(i > 0)
        def _():
            acc_ref[...] += blk_acc

    @pl.when(p == 1)
    def _phase1():
        @pl.when(i == 0)
        def _():
            m = acc_ref[0:1, :] / B
            v = acc_ref[1:2, :] / B - m * m
            s = g0_ref[...] * jax.lax.rsqrt(v + 1e-5)
            s_ref[...] = s
            t_ref[...] = b0_ref[...] - m * s

        h = h_ref[pl.ds(i * BLK, BLK), :]
        hn = (h * s_ref[...] + t_ref[...]).astype(jnp.bfloat16)
        o = jnp.dot(hn, we_ref[...].astype(jnp.bfloat16),
                    preferred_element_type=jnp.float32)
        out_ref[...] = jnp.maximum(o + be_ref[...], 0.0)


def kernel(x, W_start, b_start, bn0_g, bn0_b, W_exp, b_exp, bn_g, bn_b,
           W_end, b_end, W_dqn, b_dqn):
    del W_exp, b_exp, bn_g, bn_b, W_dqn, b_dqn
    bs = b_start.reshape(1, HID)
    g0 = bn0_g.reshape(1, HID)
    b0 = bn0_b.reshape(1, HID)
    be = b_end.reshape(1, OUT)
    return pl.pallas_call(
        _body,
        grid=(2, NB),
        in_specs=[
            pl.BlockSpec((BLK, IN_DIMS), lambda p, i: (i * (1 - p), 0)),
            pl.BlockSpec((IN_DIMS, HID), lambda p, i: (0, 0)),
            pl.BlockSpec((1, HID), lambda p, i: (0, 0)),
            pl.BlockSpec((1, HID), lambda p, i: (0, 0)),
            pl.BlockSpec((1, HID), lambda p, i: (0, 0)),
            pl.BlockSpec((HID, OUT), lambda p, i: (0, 0)),
            pl.BlockSpec((1, OUT), lambda p, i: (0, 0)),
        ],
        out_specs=pl.BlockSpec((BLK, OUT), lambda p, i: (i * p, 0)),
        out_shape=jax.ShapeDtypeStruct((B, OUT), jnp.float32),
        scratch_shapes=[
            pltpu.VMEM((B, HID), jnp.float32),
            pltpu.VMEM((2, HID), jnp.float32),
            pltpu.VMEM((1, HID), jnp.float32),
            pltpu.VMEM((1, HID), jnp.float32),
        ],
        compiler_params=pltpu.CompilerParams(
            dimension_semantics=("arbitrary", "arbitrary")),
    )(x, W_start, bs, g0, b0, W_end, be)
